# baseline (speedup 1.0000x reference)
import functools

import jax
import jax.numpy as jnp
from jax import lax
from jax.experimental import pallas as pl
from jax.experimental.pallas import tpu as pltpu
from jax.experimental.pallas import tpu_sc as plsc

_N_CODES = 100000
_D = 128
_B = 16384

_info = plsc.get_sparse_core_info()
_NC = _info.num_cores
_NS = _info.num_subcores
_NW = _NC * _NS
_B_PER_W = _B // _NW
_CHUNK = 128
_N_CHUNKS = _B_PER_W // _CHUNK


def _make_kernel():
    mesh = plsc.VectorSubcoreMesh(core_axis_name="c", subcore_axis_name="s")

    @functools.partial(
        pl.kernel,
        mesh=mesh,
        out_type=(
            jax.ShapeDtypeStruct((_B, _D), jnp.float32),
            jax.ShapeDtypeStruct((_B, _D), jnp.float32),
        ),
        scratch_types=(
            [pltpu.VMEM((_N_CHUNKS, _CHUNK), jnp.int32)]
            + [pltpu.VMEM((_CHUNK, _D), jnp.float32)] * 7
            + [pltpu.SemaphoreType.DMA] * 14
        ),
    )
    def k(ids_hbm, shape_hbm, tex_hbm, zs_hbm, zt_hbm, idx_v, *scr):
        wid = lax.axis_index("s") * _NC + lax.axis_index("c")
        base = wid * _B_PER_W
        bufs = scr[:7]
        gsem = scr[7:14]
        wsem = scr[14:21]
        pltpu.sync_copy(ids_hbm.at[wid], idx_v)
        plan = [(0, 0, 128), (1, 0, 128), (2, 0, 128), (3, 0, 64), (3, 64, 64)]
        tasks = []
        for c, off, n in plan:
            tasks.append((shape_hbm, zs_hbm, c, off, n))
            tasks.append((tex_hbm, zt_hbm, c, off, n))
        nt = len(tasks)
        nbuf = len(bufs)
        gcps = [None] * nt
        wcps = [None] * nt

        def fire_gather(i, b):
            tbl, _, c, off, n = tasks[i]
            idx = idx_v.at[c] if n == _CHUNK else idx_v.at[c, pl.ds(off, n)]
            dst = bufs[b] if n == _CHUNK else bufs[b].at[pl.ds(0, n)]
            return pltpu.async_copy(tbl.at[idx], dst, gsem[b])

        for i in range(nbuf):
            gcps[i] = fire_gather(i, i)
        for i in range(nt):
            _, out, c, off, n = tasks[i]
            b = i % nbuf
            if i >= nbuf:
                wcps[i - nbuf].wait()
                gcps[i] = fire_gather(i, b)
            gcps[i].wait()
            src = bufs[b] if n == _CHUNK else bufs[b].at[pl.ds(0, n)]
            wcps[i] = pltpu.async_copy(
                src, out.at[pl.ds(base + c * _CHUNK + off, n)], wsem[b])
        for i in range(nt - nbuf, nt):
            wcps[i].wait()

    return k


_gather2 = _make_kernel()


def kernel(object_ids, shape_table, texture_table):
    ids = object_ids.astype(jnp.int32).reshape(_NW, _N_CHUNKS, _CHUNK)
    z_s, z_t = _gather2(ids, shape_table, texture_table)
    return (z_s, z_t)


# speedup vs baseline: 1.0505x; 1.0505x over previous
import functools

import jax
import jax.numpy as jnp
from jax import lax
from jax.experimental import pallas as pl
from jax.experimental.pallas import tpu as pltpu
from jax.experimental.pallas import tpu_sc as plsc

_N_CODES = 100000
_D = 128
_B = 16384

_info = plsc.get_sparse_core_info()
_NC = _info.num_cores
_NS = _info.num_subcores
_NW = _NC * _NS
_B_PER_W = _B // _NW
_CHUNK = 128
_N_CHUNKS = _B_PER_W // _CHUNK


def _make_kernel():
    mesh = plsc.VectorSubcoreMesh(core_axis_name="c", subcore_axis_name="s")

    @functools.partial(
        pl.kernel,
        mesh=mesh,
        out_type=(
            jax.ShapeDtypeStruct((_B, _D), jnp.float32),
            jax.ShapeDtypeStruct((_B, _D), jnp.float32),
        ),
        scratch_types=(
            [pltpu.VMEM((_N_CHUNKS, _CHUNK), jnp.int32)]
            + [pltpu.VMEM((_CHUNK, _D), jnp.float32)] * 7
            + [pltpu.SemaphoreType.DMA] * 14
        ),
    )
    def k(ids_hbm, shape_hbm, tex_hbm, zs_hbm, zt_hbm, idx_v, *scr):
        wid = lax.axis_index("s") * _NC + lax.axis_index("c")
        base = wid * _B_PER_W
        bufs = scr[:7]
        gsem = scr[7:14]
        wsem = scr[14:21]
        pltpu.sync_copy(ids_hbm.at[wid], idx_v)
        tasks = [(shape_hbm, zs_hbm, c) for c in range(_N_CHUNKS)] + [
            (tex_hbm, zt_hbm, c) for c in range(_N_CHUNKS)]
        nt = len(tasks)
        nbuf = len(bufs)
        gcps = [None] * nt
        wcps = [None] * nt
        for i in range(nbuf):
            tbl, _, c = tasks[i]
            gcps[i] = pltpu.async_copy(tbl.at[idx_v.at[c]], bufs[i], gsem[i])
        for i in range(nt):
            _, out, c = tasks[i]
            b = i % nbuf
            if i >= nbuf:
                tbl, _, c_i = tasks[i]
                wcps[b].wait()
                gcps[i] = pltpu.async_copy(
                    tbl.at[idx_v.at[c_i]], bufs[b], gsem[b])
            gcps[i].wait()
            wcps[i] = pltpu.async_copy(
                bufs[b], out.at[pl.ds(base + c * _CHUNK, _CHUNK)], wsem[b])
        for i in range(nt - nbuf, nt):
            wcps[i].wait()

    return k


_gather2 = _make_kernel()


def kernel(object_ids, shape_table, texture_table):
    ids = object_ids.astype(jnp.int32).reshape(_NW, _N_CHUNKS, _CHUNK)
    z_s, z_t = _gather2(ids, shape_table, texture_table)
    return (z_s, z_t)


# speedup vs baseline: 1.0551x; 1.0044x over previous
import functools

import jax
import jax.numpy as jnp
from jax import lax
from jax.experimental import pallas as pl
from jax.experimental.pallas import tpu as pltpu
from jax.experimental.pallas import tpu_sc as plsc

_N_CODES = 100000
_D = 128
_B = 16384

_info = plsc.get_sparse_core_info()
_NC = _info.num_cores
_NS = _info.num_subcores
_NW = _NC * _NS
_B_PER_W = _B // _NW
_CHUNK = 128
_N_CHUNKS = _B_PER_W // _CHUNK


def _make_kernel():
    mesh = plsc.VectorSubcoreMesh(core_axis_name="c", subcore_axis_name="s")

    @functools.partial(
        pl.kernel,
        mesh=mesh,
        out_type=(
            jax.ShapeDtypeStruct((_B, _D), jnp.float32),
            jax.ShapeDtypeStruct((_B, _D), jnp.float32),
        ),
        scratch_types=(
            [pltpu.VMEM((_N_CHUNKS, _CHUNK), jnp.int32)]
            + [pltpu.VMEM((_CHUNK, _D), jnp.float32)] * 7
            + [pltpu.SemaphoreType.DMA] * 14
        ),
    )
    def k(ids_hbm, shape_hbm, tex_hbm, zs_hbm, zt_hbm, idx_v, *scr):
        wid = lax.axis_index("s") * _NC + lax.axis_index("c")
        base = wid * _B_PER_W
        bufs = scr[:7]
        gsem = scr[7:14]
        wsem = scr[14:21]
        pltpu.sync_copy(ids_hbm.at[wid], idx_v)
        tasks = [
            (shape_hbm, zs_hbm, 0), (tex_hbm, zt_hbm, 0),
            (shape_hbm, zs_hbm, 1), (tex_hbm, zt_hbm, 1),
            (shape_hbm, zs_hbm, 2), (tex_hbm, zt_hbm, 2),
            (shape_hbm, zs_hbm, 3),
        ]
        gcps = [None] * 7
        wcps = [None] * 7
        for i in range(7):
            tbl, _, c = tasks[i]
            gcps[i] = pltpu.async_copy(tbl.at[idx_v.at[c]], bufs[i], gsem[i])

        def fire_write(i):
            _, out, c = tasks[i]
            gcps[i].wait()
            return pltpu.async_copy(
                bufs[i], out.at[pl.ds(base + c * _CHUNK, _CHUNK)], wsem[i])

        def fire_half_gather(b, off):
            return pltpu.async_copy(
                tex_hbm.at[idx_v.at[3, pl.ds(off, 64)]],
                bufs[b].at[pl.ds(0, 64)], gsem[b])

        wcps[0] = fire_write(0)
        wcps[1] = fire_write(1)
        wcps[0].wait()
        g7a = fire_half_gather(0, 0)
        wcps[1].wait()
        g7b = fire_half_gather(1, 64)
        for i in range(2, 7):
            wcps[i] = fire_write(i)
        g7a.wait()
        w7a = pltpu.async_copy(
            bufs[0].at[pl.ds(0, 64)],
            zt_hbm.at[pl.ds(base + 3 * _CHUNK, 64)], wsem[0])
        g7b.wait()
        w7b = pltpu.async_copy(
            bufs[1].at[pl.ds(0, 64)],
            zt_hbm.at[pl.ds(base + 3 * _CHUNK + 64, 64)], wsem[1])
        for i in range(2, 7):
            wcps[i].wait()
        w7a.wait()
        w7b.wait()

    return k


_gather2 = _make_kernel()


def kernel(object_ids, shape_table, texture_table):
    ids = object_ids.astype(jnp.int32).reshape(_NW, _N_CHUNKS, _CHUNK)
    z_s, z_t = _gather2(ids, shape_table, texture_table)
    return (z_s, z_t)


# speedup vs baseline: 1.0615x; 1.0060x over previous
import functools

import jax
import jax.numpy as jnp
from jax import lax
from jax.experimental import pallas as pl
from jax.experimental.pallas import tpu as pltpu
from jax.experimental.pallas import tpu_sc as plsc

_N_CODES = 100000
_D = 128
_B = 16384

_info = plsc.get_sparse_core_info()
_NC = _info.num_cores
_NS = _info.num_subcores
_NW = _NC * _NS
_B_PER_W = _B // _NW
_CHUNK = 128
_N_CHUNKS = _B_PER_W // _CHUNK


def _make_kernel():
    mesh = plsc.VectorSubcoreMesh(core_axis_name="c", subcore_axis_name="s")

    @functools.partial(
        pl.kernel,
        mesh=mesh,
        out_type=(
            jax.ShapeDtypeStruct((_B, _D), jnp.float32),
            jax.ShapeDtypeStruct((_B, _D), jnp.float32),
        ),
        scratch_types=(
            [pltpu.VMEM((_N_CHUNKS, _CHUNK), jnp.int32)]
            + [pltpu.VMEM((7, _CHUNK, _D), jnp.float32)]
            + [pltpu.SemaphoreType.DMA((7,)), pltpu.SemaphoreType.DMA((7,))]
        ),
    )
    def k(ids_hbm, shape_hbm, tex_hbm, zs_hbm, zt_hbm, idx_v, bufs7, gs7, ws7):
        wid = lax.axis_index("s") * _NC + lax.axis_index("c")
        base = wid * _B_PER_W
        bufs = [bufs7.at[i] for i in range(7)]
        gsem = [gs7.at[i] for i in range(7)]
        wsem = [ws7.at[i] for i in range(7)]
        pltpu.sync_copy(ids_hbm.at[wid], idx_v)
        tasks = [
            (shape_hbm, zs_hbm, 0), (tex_hbm, zt_hbm, 0),
            (shape_hbm, zs_hbm, 1), (tex_hbm, zt_hbm, 1),
            (shape_hbm, zs_hbm, 2), (tex_hbm, zt_hbm, 2),
            (shape_hbm, zs_hbm, 3),
        ]
        gcps = [None] * 7
        wcps = [None] * 7
        for i in range(7):
            tbl, _, c = tasks[i]
            gcps[i] = pltpu.async_copy(tbl.at[idx_v.at[c]], bufs[i], gsem[i])

        def fire_write(i):
            _, out, c = tasks[i]
            gcps[i].wait()
            return pltpu.async_copy(
                bufs[i], out.at[pl.ds(base + c * _CHUNK, _CHUNK)], wsem[i])

        def fire_half_gather(b, off):
            return pltpu.async_copy(
                tex_hbm.at[idx_v.at[3, pl.ds(off, 64)]],
                bufs[b].at[pl.ds(0, 64)], gsem[b])

        wcps[0] = fire_write(0)
        wcps[1] = fire_write(1)
        wcps[0].wait()
        g7a = fire_half_gather(0, 0)
        wcps[1].wait()
        g7b = fire_half_gather(1, 64)
        for i in range(2, 7):
            wcps[i] = fire_write(i)
        g7a.wait()
        w7a = pltpu.async_copy(
            bufs[0].at[pl.ds(0, 64)],
            zt_hbm.at[pl.ds(base + 3 * _CHUNK, 64)], wsem[0])
        g7b.wait()
        w7b = pltpu.async_copy(
            bufs[1].at[pl.ds(0, 64)],
            zt_hbm.at[pl.ds(base + 3 * _CHUNK + 64, 64)], wsem[1])
        for i in range(2, 7):
            wcps[i].wait()
        w7a.wait()
        w7b.wait()

    return k


_gather2 = _make_kernel()


def kernel(object_ids, shape_table, texture_table):
    ids = object_ids.astype(jnp.int32).reshape(_NW, _N_CHUNKS, _CHUNK)
    z_s, z_t = _gather2(ids, shape_table, texture_table)
    return (z_s, z_t)
